# initial kernel scaffold (unmeasured)
import jax
import jax.numpy as jnp
from jax import lax
from jax.experimental import pallas as pl
from jax.experimental.pallas import tpu as pltpu

T_LOC = 1024
D = 1024
F = 4096
E_LOC = 8
E = 16
F_BLK = 1024
N_F = F // F_BLK
T_ALL = 2 * T_LOC


def _peer():
    return (lax.axis_index("x"), 1 - lax.axis_index("y"), lax.axis_index("z"))


def _peer_barrier():
    peer = _peer()
    barrier = pltpu.get_barrier_semaphore()
    pl.semaphore_signal(
        barrier, inc=1, device_id=peer, device_id_type=pl.DeviceIdType.MESH
    )
    pl.semaphore_wait(barrier, 1)
    return peer


def _exchange(x, router):

    def body(x_ref, r_ref, xall_ref, rfull_ref, sems):
        peer = _peer_barrier()

        xall_ref[pl.ds(0, T_LOC), :] = x_ref[...]
        rfull_ref[0] = r_ref[...]

        rdma_x = pltpu.make_async_remote_copy(
            src_ref=x_ref,
            dst_ref=xall_ref.at[pl.ds(T_LOC, T_LOC), :],
            send_sem=sems.at[0],
            recv_sem=sems.at[1],
            device_id=peer,
            device_id_type=pl.DeviceIdType.MESH,
        )
        rdma_r = pltpu.make_async_remote_copy(
            src_ref=r_ref,
            dst_ref=rfull_ref.at[1],
            send_sem=sems.at[2],
            recv_sem=sems.at[3],
            device_id=peer,
            device_id_type=pl.DeviceIdType.MESH,
        )
        rdma_x.start()
        rdma_r.start()
        rdma_x.wait()
        rdma_r.wait()

    return pl.pallas_call(
        body,
        out_shape=(
            jax.ShapeDtypeStruct((T_ALL, D), jnp.float32),
            jax.ShapeDtypeStruct((2, D, E_LOC), jnp.float32),
        ),
        in_specs=[
            pl.BlockSpec(memory_space=pltpu.VMEM),
            pl.BlockSpec(memory_space=pltpu.VMEM),
        ],
        out_specs=(
            pl.BlockSpec(memory_space=pltpu.VMEM),
            pl.BlockSpec(memory_space=pltpu.VMEM),
        ),
        scratch_shapes=[pltpu.SemaphoreType.DMA((4,))],
        compiler_params=pltpu.CompilerParams(collective_id=0),
    )(x, router)


def _moe(x_all, rfull, W1, W2):

    def body(xall_ref, rfull_ref, w1_ref, w2_ref, out_ref, wmat_ref):
        e = pl.program_id(0)
        f = pl.program_id(1)

        @pl.when(jnp.logical_and(e == 0, f == 0))
        def _():
            xa = xall_ref[...]
            g0 = jnp.dot(xa, rfull_ref[0], preferred_element_type=jnp.float32)
            g1 = jnp.dot(xa, rfull_ref[1], preferred_element_type=jnp.float32)
            gates = jnp.concatenate([g0, g1], axis=1)
            cols = lax.broadcasted_iota(jnp.int32, (T_ALL, E), 1)
            m1 = jnp.max(gates, axis=1, keepdims=True)
            i1 = jnp.min(jnp.where(gates == m1, cols, E), axis=1, keepdims=True)
            masked = jnp.where(cols == i1, -jnp.inf, gates)
            m2 = jnp.max(masked, axis=1, keepdims=True)
            i2 = jnp.min(jnp.where(masked == m2, cols, E), axis=1, keepdims=True)
            w_top = 1.0 / (1.0 + jnp.exp(m2 - m1))
            wmat_ref[...] = jnp.where(cols == i1, w_top, 0.0) + jnp.where(
                cols == i2, 1.0 - w_top, 0.0
            )
            out_ref[...] = jnp.zeros_like(out_ref)

        h = jnp.maximum(
            jnp.dot(xall_ref[...], w1_ref[0], preferred_element_type=jnp.float32),
            0.0,
        )
        p = jnp.dot(h, w2_ref[0], preferred_element_type=jnp.float32)
        cols = lax.broadcasted_iota(jnp.int32, (T_ALL, E), 1)
        wcol = jnp.sum(
            jnp.where(cols == e, wmat_ref[...], 0.0), axis=1, keepdims=True
        )
        out_ref[...] += p * wcol

    return pl.pallas_call(
        body,
        grid=(E_LOC, N_F),
        out_shape=jax.ShapeDtypeStruct((T_ALL, D), jnp.float32),
        in_specs=[
            pl.BlockSpec((T_ALL, D), lambda e, f: (0, 0)),
            pl.BlockSpec((2, D, E_LOC), lambda e, f: (0, 0, 0)),
            pl.BlockSpec((1, D, F_BLK), lambda e, f: (e, 0, f)),
            pl.BlockSpec((1, F_BLK, D), lambda e, f: (e, f, 0)),
        ],
        out_specs=pl.BlockSpec((T_ALL, D), lambda e, f: (0, 0)),
        scratch_shapes=[pltpu.VMEM((T_ALL, E), jnp.float32)],
        compiler_params=pltpu.CompilerParams(
            dimension_semantics=("arbitrary", "arbitrary"),
        ),
    )(x_all, rfull, W1, W2)


def _combine(contrib):

    def body(c_ref, out_ref, recv_buf, sems):
        peer = _peer_barrier()
        rdma = pltpu.make_async_remote_copy(
            src_ref=c_ref.at[pl.ds(T_LOC, T_LOC), :],
            dst_ref=recv_buf,
            send_sem=sems.at[0],
            recv_sem=sems.at[1],
            device_id=peer,
            device_id_type=pl.DeviceIdType.MESH,
        )
        rdma.start()
        rdma.wait()
        out_ref[...] = c_ref[pl.ds(0, T_LOC), :] + recv_buf[...]

    return pl.pallas_call(
        body,
        out_shape=jax.ShapeDtypeStruct((T_LOC, D), jnp.float32),
        in_specs=[pl.BlockSpec(memory_space=pltpu.VMEM)],
        out_specs=pl.BlockSpec(memory_space=pltpu.VMEM),
        scratch_shapes=[
            pltpu.VMEM((T_LOC, D), jnp.float32),
            pltpu.SemaphoreType.DMA((2,)),
        ],
        compiler_params=pltpu.CompilerParams(collective_id=1),
    )(contrib)


def kernel(x, router, W1, W2):
    x_all, rfull = _exchange(x, router)
    contrib = _moe(x_all, rfull, W1, W2)
    return _combine(contrib)


# baseline (device time: 438714 ns/iter reference)
import jax
import jax.numpy as jnp
from jax import lax
from jax.experimental import pallas as pl
from jax.experimental.pallas import tpu as pltpu

T_LOC = 1024
D = 1024
F = 4096
E_LOC = 8
E = 16
F_BLK = 1024
N_F = F // F_BLK
T_ALL = 2 * T_LOC


def _peer():
    return (lax.axis_index("x"), 1 - lax.axis_index("y"), lax.axis_index("z"))


def _peer_barrier():
    peer = _peer()
    barrier = pltpu.get_barrier_semaphore()
    pl.semaphore_signal(
        barrier, inc=1, device_id=peer, device_id_type=pl.DeviceIdType.MESH
    )
    pl.semaphore_wait(barrier, 1)
    return peer


def _exchange(x, router):

    def body(x_ref, r_ref, xall_ref, rfull_ref, sems):
        peer = _peer_barrier()

        xall_ref[pl.ds(0, T_LOC), :] = x_ref[...]
        rfull_ref[0] = r_ref[...]

        rdma_x = pltpu.make_async_remote_copy(
            src_ref=x_ref,
            dst_ref=xall_ref.at[pl.ds(T_LOC, T_LOC), :],
            send_sem=sems.at[0],
            recv_sem=sems.at[1],
            device_id=peer,
            device_id_type=pl.DeviceIdType.MESH,
        )
        rdma_r = pltpu.make_async_remote_copy(
            src_ref=r_ref,
            dst_ref=rfull_ref.at[1],
            send_sem=sems.at[2],
            recv_sem=sems.at[3],
            device_id=peer,
            device_id_type=pl.DeviceIdType.MESH,
        )
        rdma_x.start()
        rdma_r.start()
        rdma_x.wait()
        rdma_r.wait()

    return pl.pallas_call(
        body,
        out_shape=(
            jax.ShapeDtypeStruct((T_ALL, D), jnp.float32),
            jax.ShapeDtypeStruct((2, D, E_LOC), jnp.float32),
        ),
        in_specs=[
            pl.BlockSpec(memory_space=pltpu.VMEM),
            pl.BlockSpec(memory_space=pltpu.VMEM),
        ],
        out_specs=(
            pl.BlockSpec(memory_space=pltpu.VMEM),
            pl.BlockSpec(memory_space=pltpu.VMEM),
        ),
        scratch_shapes=[pltpu.SemaphoreType.DMA((4,))],
        compiler_params=pltpu.CompilerParams(collective_id=0),
    )(x, router)


def _moe(x_all, rfull, W1, W2):

    def body(xall_ref, rfull_ref, w1_ref, w2_ref, out_ref, wmat_ref):
        e = pl.program_id(0)
        f = pl.program_id(1)

        @pl.when(jnp.logical_and(e == 0, f == 0))
        def _():
            xa = xall_ref[...]
            g0 = jnp.dot(xa, rfull_ref[0], preferred_element_type=jnp.float32,
                         precision=lax.Precision.HIGHEST)
            g1 = jnp.dot(xa, rfull_ref[1], preferred_element_type=jnp.float32,
                         precision=lax.Precision.HIGHEST)
            gates = jnp.concatenate([g0, g1], axis=1)
            cols = lax.broadcasted_iota(jnp.int32, (T_ALL, E), 1)
            m1 = jnp.max(gates, axis=1, keepdims=True)
            i1 = jnp.min(jnp.where(gates == m1, cols, E), axis=1, keepdims=True)
            masked = jnp.where(cols == i1, -jnp.inf, gates)
            m2 = jnp.max(masked, axis=1, keepdims=True)
            i2 = jnp.min(jnp.where(masked == m2, cols, E), axis=1, keepdims=True)
            w_top = 1.0 / (1.0 + jnp.exp(m2 - m1))
            wmat_ref[...] = jnp.where(cols == i1, w_top, 0.0) + jnp.where(
                cols == i2, 1.0 - w_top, 0.0
            )
            out_ref[...] = jnp.zeros_like(out_ref)

        h = jnp.maximum(
            jnp.dot(xall_ref[...], w1_ref[0], preferred_element_type=jnp.float32),
            0.0,
        )
        p = jnp.dot(h, w2_ref[0], preferred_element_type=jnp.float32)
        cols = lax.broadcasted_iota(jnp.int32, (T_ALL, E), 1)
        wcol = jnp.sum(
            jnp.where(cols == e, wmat_ref[...], 0.0), axis=1, keepdims=True
        )
        out_ref[...] += p * wcol

    return pl.pallas_call(
        body,
        grid=(E_LOC, N_F),
        out_shape=jax.ShapeDtypeStruct((T_ALL, D), jnp.float32),
        in_specs=[
            pl.BlockSpec((T_ALL, D), lambda e, f: (0, 0)),
            pl.BlockSpec((2, D, E_LOC), lambda e, f: (0, 0, 0)),
            pl.BlockSpec((1, D, F_BLK), lambda e, f: (e, 0, f)),
            pl.BlockSpec((1, F_BLK, D), lambda e, f: (e, f, 0)),
        ],
        out_specs=pl.BlockSpec((T_ALL, D), lambda e, f: (0, 0)),
        scratch_shapes=[pltpu.VMEM((T_ALL, E), jnp.float32)],
        compiler_params=pltpu.CompilerParams(
            dimension_semantics=("arbitrary", "arbitrary"),
            vmem_limit_bytes=100 * 1024 * 1024,
        ),
    )(x_all, rfull, W1, W2)


def _combine(contrib):

    def body(c_ref, out_ref, recv_buf, sems):
        peer = _peer_barrier()
        rdma = pltpu.make_async_remote_copy(
            src_ref=c_ref.at[pl.ds(T_LOC, T_LOC), :],
            dst_ref=recv_buf,
            send_sem=sems.at[0],
            recv_sem=sems.at[1],
            device_id=peer,
            device_id_type=pl.DeviceIdType.MESH,
        )
        rdma.start()
        rdma.wait()
        out_ref[...] = c_ref[pl.ds(0, T_LOC), :] + recv_buf[...]

    return pl.pallas_call(
        body,
        out_shape=jax.ShapeDtypeStruct((T_LOC, D), jnp.float32),
        in_specs=[pl.BlockSpec(memory_space=pltpu.VMEM)],
        out_specs=pl.BlockSpec(memory_space=pltpu.VMEM),
        scratch_shapes=[
            pltpu.VMEM((T_LOC, D), jnp.float32),
            pltpu.SemaphoreType.DMA((2,)),
        ],
        compiler_params=pltpu.CompilerParams(collective_id=1),
    )(contrib)


def kernel(x, router, W1, W2):
    x_all, rfull = _exchange(x, router)
    contrib = _moe(x_all, rfull, W1, W2)
    return _combine(contrib)


# device time: 224944 ns/iter; 1.9503x vs baseline; 1.9503x over previous
import jax
import jax.numpy as jnp
from jax import lax
from jax.experimental import pallas as pl
from jax.experimental.pallas import tpu as pltpu

T_LOC = 1024
T_CHK = 512
D = 1024
F = 4096
E_LOC = 8
E = 16
F_BLK = 1024
N_F = F // F_BLK


def _idx():
    return lax.axis_index("x"), lax.axis_index("y"), lax.axis_index("z")


def _partner(axis):
    xi, yi, zi = _idx()
    if axis == "x":
        return (1 - xi, yi, zi)
    if axis == "y":
        return (xi, 1 - yi, zi)
    return (xi, yi, 1 - zi)


def _peer_barrier(peer):
    barrier = pltpu.get_barrier_semaphore()
    pl.semaphore_signal(
        barrier, inc=1, device_id=peer, device_id_type=pl.DeviceIdType.MESH
    )
    pl.semaphore_wait(barrier, 1)


def _dispatch(x, router):

    def body(x_ref, r_ref, xc_ref, rfull_ref, recv_buf, sems):
        xi, yi, zi = _idx()
        peer = (xi, 1 - yi, zi)
        _peer_barrier(peer)

        rfull_ref[0] = r_ref[...]
        row0 = zi * T_CHK
        rdma_x = pltpu.make_async_remote_copy(
            src_ref=x_ref.at[pl.ds(row0, T_CHK), :],
            dst_ref=recv_buf,
            send_sem=sems.at[0],
            recv_sem=sems.at[1],
            device_id=peer,
            device_id_type=pl.DeviceIdType.MESH,
        )
        rdma_r = pltpu.make_async_remote_copy(
            src_ref=r_ref,
            dst_ref=rfull_ref.at[1],
            send_sem=sems.at[2],
            recv_sem=sems.at[3],
            device_id=peer,
            device_id_type=pl.DeviceIdType.MESH,
        )
        rdma_x.start()
        rdma_r.start()
        rdma_x.wait()
        rdma_r.wait()

        own = x_ref[pl.ds(row0, T_CHK), :]
        xc_ref[...] = jnp.where(xi == yi, own, recv_buf[...])

    return pl.pallas_call(
        body,
        out_shape=(
            jax.ShapeDtypeStruct((T_CHK, D), jnp.float32),
            jax.ShapeDtypeStruct((2, D, E_LOC), jnp.float32),
        ),
        in_specs=[
            pl.BlockSpec(memory_space=pltpu.VMEM),
            pl.BlockSpec(memory_space=pltpu.VMEM),
        ],
        out_specs=(
            pl.BlockSpec(memory_space=pltpu.VMEM),
            pl.BlockSpec(memory_space=pltpu.VMEM),
        ),
        scratch_shapes=[
            pltpu.VMEM((T_CHK, D), jnp.float32),
            pltpu.SemaphoreType.DMA((4,)),
        ],
        compiler_params=pltpu.CompilerParams(collective_id=0),
    )(x, router)


def _moe(x_chunk, rfull, W1, W2):

    def body(xc_ref, rfull_ref, w1_ref, w2_ref, out_ref, wmat_ref):
        e = pl.program_id(0)
        f = pl.program_id(1)

        @pl.when(jnp.logical_and(e == 0, f == 0))
        def _():
            xa = xc_ref[...]
            g0 = jnp.dot(xa, rfull_ref[0], preferred_element_type=jnp.float32,
                         precision=lax.Precision.HIGHEST)
            g1 = jnp.dot(xa, rfull_ref[1], preferred_element_type=jnp.float32,
                         precision=lax.Precision.HIGHEST)
            gates = jnp.concatenate([g0, g1], axis=1)
            cols = lax.broadcasted_iota(jnp.int32, (T_CHK, E), 1)
            m1 = jnp.max(gates, axis=1, keepdims=True)
            i1 = jnp.min(jnp.where(gates == m1, cols, E), axis=1, keepdims=True)
            masked = jnp.where(cols == i1, -jnp.inf, gates)
            m2 = jnp.max(masked, axis=1, keepdims=True)
            i2 = jnp.min(jnp.where(masked == m2, cols, E), axis=1, keepdims=True)
            w_top = 1.0 / (1.0 + jnp.exp(m2 - m1))
            wmat_ref[...] = jnp.where(cols == i1, w_top, 0.0) + jnp.where(
                cols == i2, 1.0 - w_top, 0.0
            )
            out_ref[...] = jnp.zeros_like(out_ref)

        h = jnp.maximum(
            jnp.dot(xc_ref[...], w1_ref[0], preferred_element_type=jnp.float32),
            0.0,
        )
        p = jnp.dot(h, w2_ref[0], preferred_element_type=jnp.float32)
        cols = lax.broadcasted_iota(jnp.int32, (T_CHK, E), 1)
        wcol = jnp.sum(
            jnp.where(cols == e, wmat_ref[...], 0.0), axis=1, keepdims=True
        )
        out_ref[...] += p * wcol

    return pl.pallas_call(
        body,
        grid=(E_LOC, N_F),
        out_shape=jax.ShapeDtypeStruct((T_CHK, D), jnp.float32),
        in_specs=[
            pl.BlockSpec((T_CHK, D), lambda e, f: (0, 0)),
            pl.BlockSpec((2, D, E_LOC), lambda e, f: (0, 0, 0)),
            pl.BlockSpec((1, D, F_BLK), lambda e, f: (e, 0, f)),
            pl.BlockSpec((1, F_BLK, D), lambda e, f: (e, f, 0)),
        ],
        out_specs=pl.BlockSpec((T_CHK, D), lambda e, f: (0, 0)),
        scratch_shapes=[pltpu.VMEM((T_CHK, E), jnp.float32)],
        compiler_params=pltpu.CompilerParams(
            dimension_semantics=("arbitrary", "arbitrary"),
            vmem_limit_bytes=100 * 1024 * 1024,
        ),
    )(x_chunk, rfull, W1, W2)


def _pair_exchange(val, axis, cid, combine, out_rows):

    def body(v_ref, out_ref, recv_buf, sems):
        idx = _idx()
        peer = _partner(axis)
        _peer_barrier(peer)
        rdma = pltpu.make_async_remote_copy(
            src_ref=v_ref,
            dst_ref=recv_buf,
            send_sem=sems.at[0],
            recv_sem=sems.at[1],
            device_id=peer,
            device_id_type=pl.DeviceIdType.MESH,
        )
        rdma.start()
        rdma.wait()
        out_ref[...] = combine(v_ref[...], recv_buf[...], idx)

    rows = val.shape[0]
    return pl.pallas_call(
        body,
        out_shape=jax.ShapeDtypeStruct((out_rows, D), jnp.float32),
        in_specs=[pl.BlockSpec(memory_space=pltpu.VMEM)],
        out_specs=pl.BlockSpec(memory_space=pltpu.VMEM),
        scratch_shapes=[
            pltpu.VMEM((rows, D), jnp.float32),
            pltpu.SemaphoreType.DMA((2,)),
        ],
        compiler_params=pltpu.CompilerParams(collective_id=cid),
    )(val)


def kernel(x, router, W1, W2):
    x_chunk, rfull = _dispatch(x, router)
    contrib = _moe(x_chunk, rfull, W1, W2)
    csum = _pair_exchange(
        contrib, "y", 1, lambda m, r, idx: m + r, T_CHK
    )
    keep = _pair_exchange(
        csum, "x", 2,
        lambda m, r, idx: jnp.where(idx[0] == idx[1], m, r), T_CHK
    )
    out = _pair_exchange(
        keep, "z", 3,
        lambda m, r, idx: jnp.where(
            idx[2] == 0,
            jnp.concatenate([m, r], axis=0),
            jnp.concatenate([r, m], axis=0),
        ),
        T_LOC,
    )
    return out


# device time: 222397 ns/iter; 1.9727x vs baseline; 1.0115x over previous
import jax
import jax.numpy as jnp
from jax import lax
from jax.experimental import pallas as pl
from jax.experimental.pallas import tpu as pltpu

T_LOC = 1024
T_CHK = 512
D = 1024
F = 4096
E_LOC = 8
E = 16
F_BLK = 1024
N_F = F // F_BLK


def _idx():
    return lax.axis_index("x"), lax.axis_index("y"), lax.axis_index("z")


def _rdma(src, dst, sems, s, peer):
    return pltpu.make_async_remote_copy(
        src_ref=src,
        dst_ref=dst,
        send_sem=sems.at[s],
        recv_sem=sems.at[s + 1],
        device_id=peer,
        device_id_type=pl.DeviceIdType.MESH,
    )


def kernel(x, router, W1, W2):
    def body(x_ref, r_ref, w1_ref, w2_ref, out_ref,
             xc, xrecv, rfull, wmat, contrib, crecv, csum, xkrecv, zrecv,
             sems):
        e = pl.program_id(0)
        f = pl.program_id(1)
        xi, yi, zi = _idx()
        ypeer = (xi, 1 - yi, zi)
        xpeer = (1 - xi, yi, zi)
        zpeer = (xi, yi, 1 - zi)

        @pl.when(jnp.logical_and(e == 0, f == 0))
        def _():
            barrier = pltpu.get_barrier_semaphore()
            for p in (ypeer, xpeer, zpeer):
                pl.semaphore_signal(
                    barrier, inc=1, device_id=p,
                    device_id_type=pl.DeviceIdType.MESH,
                )
            pl.semaphore_wait(barrier, 3)

            rfull[0] = r_ref[...]
            row0 = zi * T_CHK
            rdma_x = _rdma(x_ref.at[pl.ds(row0, T_CHK), :], xrecv, sems, 0, ypeer)
            rdma_r = _rdma(r_ref, rfull.at[1], sems, 2, ypeer)
            rdma_x.start()
            rdma_r.start()
            rdma_x.wait()
            rdma_r.wait()
            xc[...] = jnp.where(xi == yi, x_ref[pl.ds(row0, T_CHK), :], xrecv[...])

            xa = xc[...]
            g0 = jnp.dot(xa, rfull[0], preferred_element_type=jnp.float32,
                         precision=lax.Precision.HIGHEST)
            g1 = jnp.dot(xa, rfull[1], preferred_element_type=jnp.float32,
                         precision=lax.Precision.HIGHEST)
            gates = jnp.concatenate([g0, g1], axis=1)
            cols = lax.broadcasted_iota(jnp.int32, (T_CHK, E), 1)
            m1 = jnp.max(gates, axis=1, keepdims=True)
            i1 = jnp.min(jnp.where(gates == m1, cols, E), axis=1, keepdims=True)
            masked = jnp.where(cols == i1, -jnp.inf, gates)
            m2 = jnp.max(masked, axis=1, keepdims=True)
            i2 = jnp.min(jnp.where(masked == m2, cols, E), axis=1, keepdims=True)
            w_top = 1.0 / (1.0 + jnp.exp(m2 - m1))
            wmat[...] = jnp.where(cols == i1, w_top, 0.0) + jnp.where(
                cols == i2, 1.0 - w_top, 0.0
            )
            contrib[...] = jnp.zeros_like(contrib)

        h = jnp.maximum(
            jnp.dot(xc[...], w1_ref[0], preferred_element_type=jnp.float32),
            0.0,
        )
        p = jnp.dot(h, w2_ref[0], preferred_element_type=jnp.float32)
        cols = lax.broadcasted_iota(jnp.int32, (T_CHK, E), 1)
        wcol = jnp.sum(
            jnp.where(cols == e, wmat[...], 0.0), axis=1, keepdims=True
        )
        contrib[...] += p * wcol

        @pl.when(jnp.logical_and(e == E_LOC - 1, f == N_F - 1))
        def _():
            rdma3 = _rdma(contrib, crecv, sems, 4, ypeer)
            rdma3.start()
            rdma3.wait()
            csum[...] = contrib[...] + crecv[...]
            rdma4 = _rdma(csum, xkrecv, sems, 6, xpeer)
            rdma4.start()
            rdma4.wait()
            contrib[...] = jnp.where(xi == yi, csum[...], xkrecv[...])
            rdma5 = _rdma(contrib, zrecv, sems, 8, zpeer)
            rdma5.start()
            rdma5.wait()
            out_ref[...] = jnp.where(
                zi == 0,
                jnp.concatenate([contrib[...], zrecv[...]], axis=0),
                jnp.concatenate([zrecv[...], contrib[...]], axis=0),
            )

    return pl.pallas_call(
        body,
        grid=(E_LOC, N_F),
        out_shape=jax.ShapeDtypeStruct((T_LOC, D), jnp.float32),
        in_specs=[
            pl.BlockSpec(memory_space=pltpu.VMEM),
            pl.BlockSpec(memory_space=pltpu.VMEM),
            pl.BlockSpec((1, D, F_BLK), lambda e, f: (e, 0, f)),
            pl.BlockSpec((1, F_BLK, D), lambda e, f: (e, f, 0)),
        ],
        out_specs=pl.BlockSpec((T_LOC, D), lambda e, f: (0, 0)),
        scratch_shapes=[
            pltpu.VMEM((T_CHK, D), jnp.float32),
            pltpu.VMEM((T_CHK, D), jnp.float32),
            pltpu.VMEM((2, D, E_LOC), jnp.float32),
            pltpu.VMEM((T_CHK, E), jnp.float32),
            pltpu.VMEM((T_CHK, D), jnp.float32),
            pltpu.VMEM((T_CHK, D), jnp.float32),
            pltpu.VMEM((T_CHK, D), jnp.float32),
            pltpu.VMEM((T_CHK, D), jnp.float32),
            pltpu.VMEM((T_CHK, D), jnp.float32),
            pltpu.SemaphoreType.DMA((10,)),
        ],
        compiler_params=pltpu.CompilerParams(
            dimension_semantics=("arbitrary", "arbitrary"),
            vmem_limit_bytes=100 * 1024 * 1024,
            collective_id=0,
        ),
    )(x, router, W1, W2)
